# baseline (speedup 1.0000x reference)
import functools

import jax
import jax.numpy as jnp
import numpy as np
from jax.experimental import pallas as pl
from jax.experimental.pallas import tpu as pltpu

HIDDEN = 2048
FFN = 8192
NHEAD = 16
HEAD_DIM = HIDDEN // NHEAD
EPS = 1e-5
_INV_SQRT_HD = 1.0 / float(np.sqrt(HEAD_DIM))

_VMEM_LIMIT = 58 * 1024 * 1024


def _cp(ndims):
    return pltpu.CompilerParams(
        dimension_semantics=("parallel",) * ndims,
        vmem_limit_bytes=_VMEM_LIMIT,
    )


def _mm_body(x_ref, w_ref, b_ref, o_ref, *, relu):
    acc = jax.lax.dot_general(
        x_ref[...].astype(jnp.bfloat16), w_ref[...].astype(jnp.bfloat16),
        (((1,), (1,)), ((), ())),
        preferred_element_type=jnp.float32)
    acc = acc + b_ref[...]
    if relu:
        acc = jnp.maximum(acc, 0.0)
    o_ref[...] = acc.astype(o_ref.dtype)


def _mm(x, w, b, bm, bn, relu=False, out_dtype=jnp.float32):
    M, K = x.shape
    xs = [x]
    x_specs = [pl.BlockSpec((bm, K), lambda n, m: (m, 0))]
    body = functools.partial(_mm_body, relu=relu)
    N, _ = w.shape
    grid = (N // bn, M // bm)
    return pl.pallas_call(
        body,
        grid=grid,
        in_specs=[
            *x_specs,
            pl.BlockSpec((bn, K), lambda n, m: (n, 0)),
            pl.BlockSpec((1, bn), lambda n, m: (0, n)),
        ],
        out_specs=pl.BlockSpec((bm, bn), lambda n, m: (m, n)),
        out_shape=jax.ShapeDtypeStruct((M, N), out_dtype),
        compiler_params=_cp(2),
    )(*xs, w, b.reshape(1, N))


def _attn_body(q_ref, k_ref, v_ref, o_ref):
    q = q_ref[...] * jnp.bfloat16(_INV_SQRT_HD)
    s = jax.lax.dot_general(
        q, k_ref[...],
        (((1,), (1,)), ((), ())), preferred_element_type=jnp.float32)
    p = jnp.exp(s)
    l = jnp.sum(p, axis=1, keepdims=True)
    ctx = jax.lax.dot_general(
        p.astype(jnp.bfloat16), v_ref[...],
        (((1,), (0,)), ((), ())), preferred_element_type=jnp.float32)
    o_ref[...] = (ctx * (1.0 / l)).astype(jnp.bfloat16)


def _attention(qkv, T, bq):
    grid = (NHEAD, T // bq)
    return pl.pallas_call(
        _attn_body,
        grid=grid,
        in_specs=[
            pl.BlockSpec((bq, HEAD_DIM), lambda h, i: (i, h)),
            pl.BlockSpec((T, HEAD_DIM), lambda h, i: (0, NHEAD + h)),
            pl.BlockSpec((T, HEAD_DIM), lambda h, i: (0, 2 * NHEAD + h)),
        ],
        out_specs=pl.BlockSpec((bq, HEAD_DIM), lambda h, i: (i, h)),
        out_shape=jax.ShapeDtypeStruct((T, HIDDEN), jnp.bfloat16),
        compiler_params=_cp(2),
    )(qkv, qkv, qkv)


def _layer_norm(y, g, bb):
    mu = jnp.mean(y, axis=1, keepdims=True)
    d = y - mu
    var = jnp.mean(d * d, axis=1, keepdims=True)
    inv = 1.0 / jnp.sqrt(var + EPS)
    return d * inv * g + bb


def _mm_ln_body(x_ref, w_ref, b_ref, r_ref, g_ref, bb_ref, o_ref, obf_ref):
    y = jax.lax.dot_general(
        x_ref[...].astype(jnp.bfloat16), w_ref[...].astype(jnp.bfloat16),
        (((1,), (1,)), ((), ())),
        preferred_element_type=jnp.float32)
    y = y + b_ref[...] + r_ref[...]
    out = _layer_norm(y, g_ref[...], bb_ref[...])
    o_ref[...] = out
    obf_ref[...] = out.astype(jnp.bfloat16)


def _mm_ln(x, w, b, res, g, beta, bm):
    M, K = x.shape
    N, _ = w.shape
    grid = (M // bm,)
    row = lambda m: (m, 0)
    vec = lambda m: (0, 0)
    body = _mm_ln_body
    extra_in = [res]
    extra_specs = [pl.BlockSpec((bm, N), row)]
    return pl.pallas_call(
        body,
        grid=grid,
        in_specs=[
            pl.BlockSpec((bm, K), row),
            pl.BlockSpec((N, K), vec),
            pl.BlockSpec((1, N), vec),
            *extra_specs,
            pl.BlockSpec((1, N), vec),
            pl.BlockSpec((1, N), vec),
        ],
        out_specs=[pl.BlockSpec((bm, N), row), pl.BlockSpec((bm, N), row)],
        out_shape=[jax.ShapeDtypeStruct((M, N), jnp.float32),
                   jax.ShapeDtypeStruct((M, N), jnp.bfloat16)],
        compiler_params=_cp(1),
    )(x, w, b.reshape(1, N), *extra_in, g.reshape(1, N), beta.reshape(1, N))


def _ln_body(y_ref, r_ref, g_ref, bb_ref, o_ref, obf_ref):
    out = _layer_norm(y_ref[...] + r_ref[...], g_ref[...], bb_ref[...])
    o_ref[...] = out
    obf_ref[...] = out.astype(jnp.bfloat16)


def _ln(y, res, g, beta, bm):
    M, N = y.shape
    return pl.pallas_call(
        _ln_body,
        grid=(M // bm,),
        in_specs=[
            pl.BlockSpec((bm, N), lambda m: (m, 0)),
            pl.BlockSpec((bm, N), lambda m: (m, 0)),
            pl.BlockSpec((1, N), lambda m: (0, 0)),
            pl.BlockSpec((1, N), lambda m: (0, 0)),
        ],
        out_specs=[pl.BlockSpec((bm, N), lambda m: (m, 0)),
                   pl.BlockSpec((bm, N), lambda m: (m, 0))],
        out_shape=[jax.ShapeDtypeStruct((M, N), jnp.float32),
                   jax.ShapeDtypeStruct((M, N), jnp.bfloat16)],
        compiler_params=_cp(1),
    )(y, res, g.reshape(1, N), beta.reshape(1, N))


def _ln_sel_body(y_ref, r_ref, g_ref, bb_ref, m_ref, orig_ref, o_ref):
    y = y_ref[...] + r_ref[...]
    mu = jnp.mean(y, axis=1, keepdims=True)
    d = y - mu
    var = jnp.mean(d * d, axis=1, keepdims=True)
    inv = 1.0 / jnp.sqrt(var + EPS)
    ln = d * inv * g_ref[...] + bb_ref[...]
    o_ref[...] = jnp.where(m_ref[...] > 0.5, ln, orig_ref[...])


def _ln_sel(y, res, g, beta, mask_col, orig, bm):
    M, N = y.shape
    return pl.pallas_call(
        _ln_sel_body,
        grid=(M // bm,),
        in_specs=[
            pl.BlockSpec((bm, N), lambda m: (m, 0)),
            pl.BlockSpec((bm, N), lambda m: (m, 0)),
            pl.BlockSpec((1, N), lambda m: (0, 0)),
            pl.BlockSpec((1, N), lambda m: (0, 0)),
            pl.BlockSpec((bm, 1), lambda m: (m, 0)),
            pl.BlockSpec((bm, N), lambda m: (m, 0)),
        ],
        out_specs=pl.BlockSpec((bm, N), lambda m: (m, 0)),
        out_shape=jax.ShapeDtypeStruct((M, N), jnp.float32),
        compiler_params=_cp(1),
    )(y, res, g.reshape(1, N), beta.reshape(1, N), mask_col, orig)


def _encoder_layer(x, res, p, T):
    bf = jnp.bfloat16
    qkv = _mm(x, p["Wqkv"], p["bqkv"], bm=512, bn=2048, out_dtype=bf)
    ctx = _attention(qkv, T, bq=1024)
    x1, x1_bf = _mm_ln(ctx, p['Wo'], p['bo'], res,
                       p['ln1_g'], p['ln1_b'], bm=512)
    h = _mm(x1_bf, p['W1'], p['b1'], bm=512, bn=2048, relu=True, out_dtype=bf)
    y = _mm(h, p['W2'].astype(bf), p['b2'], bm=512, bn=512)
    return x1, y


def kernel(hidden_states, gate_w, gate_b, layer0, layer1):
    B, S, D = hidden_states.shape
    T = B * S
    flat = hidden_states.reshape(T, D)

    depth_logits = (hidden_states @ gate_w.T + gate_b).squeeze(-1)
    depth_probs = jax.nn.sigmoid(depth_logits)
    depth_decisions = (depth_probs > 0.5).astype(jnp.float32)
    skip_rate = 1.0 - jnp.mean(depth_decisions)
    mask_col = depth_decisions.reshape(T, 1)

    x1_a, y_a = _encoder_layer(flat, flat, layer0, T)
    x_b, x_b_bf = _ln(y_a, x1_a, layer0['ln2_g'], layer0['ln2_b'], bm=512)
    x1_b, y_b = _encoder_layer(x_b_bf, x_b, layer1, T)
    out_flat = _ln_sel(y_b, x1_b, layer1['ln2_g'], layer1['ln2_b'],
                       mask_col, flat, bm=512)

    output_states = out_flat.reshape(B, S, D)
    return (output_states, skip_rate)


# speedup vs baseline: 1.0137x; 1.0137x over previous
import functools

import jax
import jax.numpy as jnp
import numpy as np
from jax.experimental import pallas as pl
from jax.experimental.pallas import tpu as pltpu

HIDDEN = 2048
FFN = 8192
NHEAD = 16
HEAD_DIM = HIDDEN // NHEAD
EPS = 1e-5
_INV_SQRT_HD = 1.0 / float(np.sqrt(HEAD_DIM))

_VMEM_LIMIT = 58 * 1024 * 1024


def _cp(ndims):
    return pltpu.CompilerParams(
        dimension_semantics=("parallel",) * ndims,
        vmem_limit_bytes=_VMEM_LIMIT,
    )


def _mm_body(x_ref, w_ref, b_ref, o_ref, *, relu):
    acc = jax.lax.dot_general(
        x_ref[...].astype(jnp.bfloat16), w_ref[...].astype(jnp.bfloat16),
        (((1,), (1,)), ((), ())),
        preferred_element_type=jnp.float32)
    acc = acc + b_ref[...]
    if relu:
        acc = jnp.maximum(acc, 0.0)
    o_ref[...] = acc.astype(o_ref.dtype)


def _mm(x, w, b, bm, bn, relu=False, out_dtype=jnp.float32):
    M, K = x.shape
    xs = [x]
    x_specs = [pl.BlockSpec((bm, K), lambda n, m: (m, 0))]
    body = functools.partial(_mm_body, relu=relu)
    N, _ = w.shape
    grid = (N // bn, M // bm)
    return pl.pallas_call(
        body,
        grid=grid,
        in_specs=[
            *x_specs,
            pl.BlockSpec((bn, K), lambda n, m: (n, 0)),
            pl.BlockSpec((1, bn), lambda n, m: (0, n)),
        ],
        out_specs=pl.BlockSpec((bm, bn), lambda n, m: (m, n)),
        out_shape=jax.ShapeDtypeStruct((M, N), out_dtype),
        compiler_params=_cp(2),
    )(*xs, w, b.reshape(1, N))


def _attn_body(q_ref, k_ref, v_ref, o_ref):
    q = q_ref[...] * jnp.bfloat16(_INV_SQRT_HD)
    s = jax.lax.dot_general(
        q, k_ref[...],
        (((1,), (1,)), ((), ())), preferred_element_type=jnp.float32)
    p = jnp.exp(s)
    l = jnp.sum(p, axis=1, keepdims=True)
    ctx = jax.lax.dot_general(
        p.astype(jnp.bfloat16), v_ref[...],
        (((1,), (0,)), ((), ())), preferred_element_type=jnp.float32)
    o_ref[...] = (ctx * (1.0 / l)).astype(jnp.bfloat16)


def _attention(qkv, T, bq):
    grid = (NHEAD, T // bq)
    return pl.pallas_call(
        _attn_body,
        grid=grid,
        in_specs=[
            pl.BlockSpec((bq, HEAD_DIM), lambda h, i: (i, h)),
            pl.BlockSpec((T, HEAD_DIM), lambda h, i: (0, NHEAD + h)),
            pl.BlockSpec((T, HEAD_DIM), lambda h, i: (0, 2 * NHEAD + h)),
        ],
        out_specs=pl.BlockSpec((bq, HEAD_DIM), lambda h, i: (i, h)),
        out_shape=jax.ShapeDtypeStruct((T, HIDDEN), jnp.bfloat16),
        compiler_params=_cp(2),
    )(qkv, qkv, qkv)


def _layer_norm(y, g, bb):
    mu = jnp.mean(y, axis=1, keepdims=True)
    d = y - mu
    var = jnp.mean(d * d, axis=1, keepdims=True)
    inv = 1.0 / jnp.sqrt(var + EPS)
    return d * inv * g + bb


def _mm_ln_body(x_ref, w_ref, b_ref, r_ref, g_ref, bb_ref, o_ref, obf_ref):
    y = jax.lax.dot_general(
        x_ref[...].astype(jnp.bfloat16), w_ref[...].astype(jnp.bfloat16),
        (((1,), (1,)), ((), ())),
        preferred_element_type=jnp.float32)
    y = y + b_ref[...] + r_ref[...]
    out = _layer_norm(y, g_ref[...], bb_ref[...])
    o_ref[...] = out
    obf_ref[...] = out.astype(jnp.bfloat16)


def _mm_ln(x, w, b, res, g, beta, bm):
    M, K = x.shape
    N, _ = w.shape
    grid = (M // bm,)
    row = lambda m: (m, 0)
    vec = lambda m: (0, 0)
    body = _mm_ln_body
    extra_in = [res]
    extra_specs = [pl.BlockSpec((bm, N), row)]
    return pl.pallas_call(
        body,
        grid=grid,
        in_specs=[
            pl.BlockSpec((bm, K), row),
            pl.BlockSpec((N, K), vec),
            pl.BlockSpec((1, N), vec),
            *extra_specs,
            pl.BlockSpec((1, N), vec),
            pl.BlockSpec((1, N), vec),
        ],
        out_specs=[pl.BlockSpec((bm, N), row), pl.BlockSpec((bm, N), row)],
        out_shape=[jax.ShapeDtypeStruct((M, N), jnp.float32),
                   jax.ShapeDtypeStruct((M, N), jnp.bfloat16)],
        compiler_params=_cp(1),
    )(x, w, b.reshape(1, N), *extra_in, g.reshape(1, N), beta.reshape(1, N))


def _ln_body(y_ref, r_ref, g_ref, bb_ref, o_ref, obf_ref):
    out = _layer_norm(y_ref[...] + r_ref[...], g_ref[...], bb_ref[...])
    o_ref[...] = out
    obf_ref[...] = out.astype(jnp.bfloat16)


def _ln(y, res, g, beta, bm):
    M, N = y.shape
    return pl.pallas_call(
        _ln_body,
        grid=(M // bm,),
        in_specs=[
            pl.BlockSpec((bm, N), lambda m: (m, 0)),
            pl.BlockSpec((bm, N), lambda m: (m, 0)),
            pl.BlockSpec((1, N), lambda m: (0, 0)),
            pl.BlockSpec((1, N), lambda m: (0, 0)),
        ],
        out_specs=[pl.BlockSpec((bm, N), lambda m: (m, 0)),
                   pl.BlockSpec((bm, N), lambda m: (m, 0))],
        out_shape=[jax.ShapeDtypeStruct((M, N), jnp.float32),
                   jax.ShapeDtypeStruct((M, N), jnp.bfloat16)],
        compiler_params=_cp(1),
    )(y, res, g.reshape(1, N), beta.reshape(1, N))


def _ln_sel_body(y_ref, r_ref, g_ref, bb_ref, m_ref, orig_ref, o_ref):
    y = y_ref[...] + r_ref[...]
    mu = jnp.mean(y, axis=1, keepdims=True)
    d = y - mu
    var = jnp.mean(d * d, axis=1, keepdims=True)
    inv = 1.0 / jnp.sqrt(var + EPS)
    ln = d * inv * g_ref[...] + bb_ref[...]
    o_ref[...] = jnp.where(m_ref[...] > 0.5, ln, orig_ref[...])


def _ln_sel(y, res, g, beta, mask_col, orig, bm):
    M, N = y.shape
    return pl.pallas_call(
        _ln_sel_body,
        grid=(M // bm,),
        in_specs=[
            pl.BlockSpec((bm, N), lambda m: (m, 0)),
            pl.BlockSpec((bm, N), lambda m: (m, 0)),
            pl.BlockSpec((1, N), lambda m: (0, 0)),
            pl.BlockSpec((1, N), lambda m: (0, 0)),
            pl.BlockSpec((bm, 1), lambda m: (m, 0)),
            pl.BlockSpec((bm, N), lambda m: (m, 0)),
        ],
        out_specs=pl.BlockSpec((bm, N), lambda m: (m, 0)),
        out_shape=jax.ShapeDtypeStruct((M, N), jnp.float32),
        compiler_params=_cp(1),
    )(y, res, g.reshape(1, N), beta.reshape(1, N), mask_col, orig)


def _encoder_layer(x, res, p, T):
    bf = jnp.bfloat16
    qkv = _mm(x, p["Wqkv"], p["bqkv"], bm=512, bn=2048, out_dtype=bf)
    ctx = _attention(qkv, T, bq=2048)
    x1, x1_bf = _mm_ln(ctx, p['Wo'], p['bo'], res,
                       p['ln1_g'], p['ln1_b'], bm=512)
    h = _mm(x1_bf, p['W1'], p['b1'], bm=512, bn=2048, relu=True, out_dtype=bf)
    y = _mm(h, p['W2'].astype(bf), p['b2'], bm=512, bn=512)
    return x1, y


def kernel(hidden_states, gate_w, gate_b, layer0, layer1):
    B, S, D = hidden_states.shape
    T = B * S
    flat = hidden_states.reshape(T, D)

    depth_logits = (hidden_states @ gate_w.T + gate_b).squeeze(-1)
    depth_probs = jax.nn.sigmoid(depth_logits)
    depth_decisions = (depth_probs > 0.5).astype(jnp.float32)
    skip_rate = 1.0 - jnp.mean(depth_decisions)
    mask_col = depth_decisions.reshape(T, 1)

    x1_a, y_a = _encoder_layer(flat, flat, layer0, T)
    x_b, x_b_bf = _ln(y_a, x1_a, layer0['ln2_g'], layer0['ln2_b'], bm=512)
    x1_b, y_b = _encoder_layer(x_b_bf, x_b, layer1, T)
    out_flat = _ln_sel(y_b, x1_b, layer1['ln2_g'], layer1['ln2_b'],
                       mask_col, flat, bm=512)

    output_states = out_flat.reshape(B, S, D)
    return (output_states, skip_rate)
